# Initial kernel scaffold; baseline (speedup 1.0000x reference)
import jax
import jax.numpy as jnp
from jax.experimental import pallas as pl


def kernel(x, edge_index, e_ij, fW1, fb1, fW2, fb2, uW1, ub1, uW2, ub2):
    raise NotImplementedError("write your pallas kernel here")


# speedup vs baseline: 2.6374x; 2.6374x over previous
import functools

import jax
import jax.numpy as jnp
from jax import lax
from jax.experimental import pallas as pl
from jax.experimental.pallas import tpu as pltpu
from jax.experimental.pallas import tpu_sc as plsc

N = 10000
E = 320000
D = 128
NRBF = 16

NC = 2
NS = 16
NW = NC * NS
SB = 128
TS = E // SB
BASE_STREAMS = TS // NW
EXTRA = TS - BASE_STREAMS * NW
ZR = 624
ZR_TAIL = N - ZR * NS

_LOG2 = 0.6931471805599453


def _ssp(t):
    return jnp.maximum(t, 0.0) + jnp.log1p(jnp.exp(-jnp.abs(t))) - _LOG2


_BE = 2000


def _filter_body(e_ref, w1_ref, b1_ref, w2_ref, b2_ref, o_ref):
    t = jnp.dot(e_ref[...], w1_ref[...], preferred_element_type=jnp.float32)
    h = _ssp(t + b1_ref[...])
    o_ref[...] = (
        jnp.dot(h, w2_ref[...], preferred_element_type=jnp.float32)
        + b2_ref[...]
    )


def _filter_mlp(e_ij, fW1, fb1, fW2, fb2):
    grid = E // _BE
    return pl.pallas_call(
        _filter_body,
        grid=(grid,),
        in_specs=[
            pl.BlockSpec((_BE, NRBF), lambda i: (i, 0)),
            pl.BlockSpec((NRBF, D), lambda i: (0, 0)),
            pl.BlockSpec((1, D), lambda i: (0, 0)),
            pl.BlockSpec((D, D), lambda i: (0, 0)),
            pl.BlockSpec((1, D), lambda i: (0, 0)),
        ],
        out_specs=pl.BlockSpec((_BE, D), lambda i: (i, 0)),
        out_shape=jax.ShapeDtypeStruct((E, D), jnp.float32),
    )(e_ij, fW1, fb1.reshape(1, D), fW2, fb2.reshape(1, D))


def _sc_body(w_hbm, x_hbm, ei_hbm, out_hbm, src_v, dst_v, xg_v, wv, acc_sh,
             sem):
    c = lax.axis_index("c")
    s = lax.axis_index("s")
    wid = s * NC + c

    def zrow(r, _):
        for j in range(D // 16):
            xg_v[r, pl.ds(16 * j, 16)] = jnp.zeros((16,), jnp.float32)
        return 0

    lax.fori_loop(0, SB, zrow, 0)
    r0 = s * ZR
    for k in range(ZR // SB):
        pltpu.sync_copy(xg_v, acc_sh.at[pl.ds(r0 + SB * k, SB)])
    rem = ZR % SB
    if rem:
        pltpu.sync_copy(xg_v.at[pl.ds(0, rem)],
                        acc_sh.at[pl.ds(r0 + (ZR // SB) * SB, rem)])

    @pl.when(s == 0)
    def _zero_tail():
        pltpu.sync_copy(xg_v.at[pl.ds(0, ZR_TAIL)],
                        acc_sh.at[pl.ds(ZR * NS, ZR_TAIL)])

    plsc.subcore_barrier()

    start = wid * BASE_STREAMS + jnp.minimum(wid, EXTRA)
    n = BASE_STREAMS + jnp.where(wid < EXTRA, 1, 0)

    def stream_body(si, _):
        ebase = si * SB
        pltpu.sync_copy(ei_hbm.at[0, pl.ds(ebase, SB)], src_v.at[0])
        pltpu.sync_copy(ei_hbm.at[1, pl.ds(ebase, SB)], dst_v.at[0])
        pltpu.async_copy(x_hbm.at[dst_v.at[0]], xg_v, sem).wait()
        pltpu.sync_copy(w_hbm.at[pl.ds(ebase, SB)], wv)

        def mrow(r, _):
            for j in range(D // 16):
                sl = pl.ds(16 * j, 16)
                wv[r, sl] = wv[r, sl] * xg_v[r, sl]
            return 0

        lax.fori_loop(0, SB, mrow, 0)
        pltpu.sync_copy(wv, acc_sh.at[src_v.at[0]], add=True)
        return 0

    lax.fori_loop(start, start + n, stream_body, 0)
    plsc.subcore_barrier()

    pltpu.sync_copy(acc_sh.at[pl.ds(r0, ZR)], out_hbm.at[c, pl.ds(r0, ZR)])

    @pl.when(s == 0)
    def _write_tail():
        pltpu.sync_copy(acc_sh.at[pl.ds(ZR * NS, ZR_TAIL)],
                        out_hbm.at[c, pl.ds(ZR * NS, ZR_TAIL)])


def _sc_gather_scatter(w, x, ei):
    f = functools.partial(
        pl.kernel,
        out_type=jax.ShapeDtypeStruct((NC, N, D), jnp.float32),
        mesh=plsc.VectorSubcoreMesh(core_axis_name="c", subcore_axis_name="s"),
        scratch_types=[
            pltpu.VMEM((1, SB), jnp.int32),
            pltpu.VMEM((1, SB), jnp.int32),
            pltpu.VMEM((SB, D), jnp.float32),
            pltpu.VMEM((SB, D), jnp.float32),
            pltpu.VMEM_SHARED((N, D), jnp.float32),
            pltpu.SemaphoreType.DMA,
        ],
    )(_sc_body)
    return f(w, x, ei)


_BN = 2000


def _update_body(p_ref, x_ref, w1_ref, b1_ref, w2_ref, b2_ref, o_ref):
    m = p_ref[0] + p_ref[1]
    u = _ssp(jnp.dot(m, w1_ref[...], preferred_element_type=jnp.float32)
             + b1_ref[...])
    o_ref[...] = (
        x_ref[...]
        + jnp.dot(u, w2_ref[...], preferred_element_type=jnp.float32)
        + b2_ref[...]
    )


def _update_mlp(partials, x, uW1, ub1, uW2, ub2):
    grid = N // _BN
    return pl.pallas_call(
        _update_body,
        grid=(grid,),
        in_specs=[
            pl.BlockSpec((NC, _BN, D), lambda i: (0, i, 0)),
            pl.BlockSpec((_BN, D), lambda i: (i, 0)),
            pl.BlockSpec((D, D), lambda i: (0, 0)),
            pl.BlockSpec((1, D), lambda i: (0, 0)),
            pl.BlockSpec((D, D), lambda i: (0, 0)),
            pl.BlockSpec((1, D), lambda i: (0, 0)),
        ],
        out_specs=pl.BlockSpec((_BN, D), lambda i: (i, 0)),
        out_shape=jax.ShapeDtypeStruct((N, D), jnp.float32),
    )(partials, x, uW1, ub1.reshape(1, D), uW2, ub2.reshape(1, D))


def kernel(x, edge_index, e_ij, fW1, fb1, fW2, fb2, uW1, ub1, uW2, ub2):
    ei = edge_index.astype(jnp.int32)
    w = _filter_mlp(e_ij, fW1, fb1, fW2, fb2)
    partials = _sc_gather_scatter(w, x, ei)
    return _update_mlp(partials, x, uW1, ub1, uW2, ub2)


# speedup vs baseline: 3.3605x; 1.2742x over previous
import functools

import jax
import jax.numpy as jnp
from jax import lax
from jax.experimental import pallas as pl
from jax.experimental.pallas import tpu as pltpu
from jax.experimental.pallas import tpu_sc as plsc

N = 10000
E = 320000
D = 128
NRBF = 16

NC = 2
NS = 16
NW = NC * NS
SB = 128
TS = E // SB
NSTR = TS // NW
EXTRA = TS - NSTR * NW
ZR = 624
ZR_TAIL = N - ZR * NS

_LOG2 = 0.6931471805599453


def _ssp(t):
    return jnp.maximum(t, 0.0) + jnp.log1p(jnp.exp(-jnp.abs(t))) - _LOG2


_BE = 2000


def _filter_body(e_ref, w1_ref, b1_ref, w2_ref, b2_ref, o_ref):
    t = jnp.dot(e_ref[...], w1_ref[...], preferred_element_type=jnp.float32)
    h = _ssp(t + b1_ref[...])
    o_ref[...] = (
        jnp.dot(h, w2_ref[...], preferred_element_type=jnp.float32)
        + b2_ref[...]
    )


def _filter_mlp(e_ij, fW1, fb1, fW2, fb2):
    grid = E // _BE
    return pl.pallas_call(
        _filter_body,
        grid=(grid,),
        in_specs=[
            pl.BlockSpec((_BE, NRBF), lambda i: (i, 0)),
            pl.BlockSpec((NRBF, D), lambda i: (0, 0)),
            pl.BlockSpec((1, D), lambda i: (0, 0)),
            pl.BlockSpec((D, D), lambda i: (0, 0)),
            pl.BlockSpec((1, D), lambda i: (0, 0)),
        ],
        out_specs=pl.BlockSpec((_BE, D), lambda i: (i, 0)),
        out_shape=jax.ShapeDtypeStruct((E, D), jnp.float32),
    )(e_ij, fW1, fb1.reshape(1, D), fW2, fb2.reshape(1, D))


def _sc_body(w_hbm, x_hbm, ei_hbm, out_hbm,
             src0, src1, dst0, dst1, xg0, xg1, wv, acc_sh, sg0, sg1, sw0):
    c = lax.axis_index("c")
    s = lax.axis_index("s")
    wid = s * NC + c
    xg = (xg0, xg1)
    srcb = (src0, src1)
    dstb = (dst0, dst1)
    sg = (sg0, sg1)
    sbase = wid * NSTR

    def zrow(r, _):
        for j in range(D // 16):
            xg0[r, pl.ds(16 * j, 16)] = jnp.zeros((16,), jnp.float32)
        return 0

    lax.fori_loop(0, SB, zrow, 0)
    r0 = s * ZR
    for k in range(ZR // SB):
        pltpu.sync_copy(xg0, acc_sh.at[pl.ds(r0 + SB * k, SB)])
    rem = ZR % SB
    if rem:
        pltpu.sync_copy(xg0.at[pl.ds(0, rem)],
                        acc_sh.at[pl.ds(r0 + (ZR // SB) * SB, rem)])

    @pl.when(s == 0)
    def _zero_tail():
        pltpu.sync_copy(xg0.at[pl.ds(0, ZR_TAIL)],
                        acc_sh.at[pl.ds(ZR * NS, ZR_TAIL)])

    plsc.subcore_barrier()

    def eb(si):
        return (sbase + si) * SB

    def issue_gather(si, b):
        pltpu.sync_copy(ei_hbm.at[1, pl.ds(eb(si), SB)], dstb[b].at[0])
        pltpu.async_copy(x_hbm.at[dstb[b].at[0]], xg[b], sg[b])
        pltpu.sync_copy(ei_hbm.at[0, pl.ds(eb(si), SB)], srcb[b].at[0])

    def issue_w(si):
        pltpu.async_copy(w_hbm.at[pl.ds(eb(si), SB)], wv, sw0)

    def wait_gather(b):
        pltpu.make_async_copy(x_hbm.at[dstb[b].at[0]], xg[b], sg[b]).wait()

    def wait_w(si):
        pltpu.make_async_copy(w_hbm.at[pl.ds(eb(si), SB)], wv, sw0).wait()

    def compute(b):
        def mrow(r, _):
            for j in range(D // 16):
                sl = pl.ds(16 * j, 16)
                wv[r, sl] = wv[r, sl] * xg[b][r, sl]
            return 0

        lax.fori_loop(0, SB, mrow, 0)

    def scatter(b):
        pltpu.sync_copy(wv, acc_sh.at[srcb[b].at[0]], add=True)

    issue_gather(0, 0)
    issue_gather(1, 1)
    issue_w(0)

    def pair(g, _):
        for b in (0, 1):
            si = 2 * g + b
            wait_gather(b)
            wait_w(si)
            compute(b)
            scatter(b)

            @pl.when(si + 1 < NSTR)
            def _next_w():
                issue_w(si + 1)

            @pl.when(si + 2 < NSTR)
            def _next_g():
                issue_gather(si + 2, b)

        return 0

    lax.fori_loop(0, NSTR // 2, pair, 0)

    @pl.when(wid < EXTRA)
    def _extra_stream():
        ebx = (NSTR * NW + wid) * SB
        pltpu.sync_copy(ei_hbm.at[1, pl.ds(ebx, SB)], dst0.at[0])
        pltpu.async_copy(x_hbm.at[dst0.at[0]], xg0, sg0).wait()
        pltpu.sync_copy(ei_hbm.at[0, pl.ds(ebx, SB)], src0.at[0])
        pltpu.async_copy(w_hbm.at[pl.ds(ebx, SB)], wv, sw0).wait()
        compute(0)
        scatter(0)

    plsc.subcore_barrier()

    pltpu.sync_copy(acc_sh.at[pl.ds(r0, ZR)], out_hbm.at[c, pl.ds(r0, ZR)])

    @pl.when(s == 0)
    def _write_tail():
        pltpu.sync_copy(acc_sh.at[pl.ds(ZR * NS, ZR_TAIL)],
                        out_hbm.at[c, pl.ds(ZR * NS, ZR_TAIL)])


def _sc_gather_scatter(w, x, ei):
    f = functools.partial(
        pl.kernel,
        out_type=jax.ShapeDtypeStruct((NC, N, D), jnp.float32),
        mesh=plsc.VectorSubcoreMesh(core_axis_name="c", subcore_axis_name="s"),
        scratch_types=[
            pltpu.VMEM((1, SB), jnp.int32),
            pltpu.VMEM((1, SB), jnp.int32),
            pltpu.VMEM((1, SB), jnp.int32),
            pltpu.VMEM((1, SB), jnp.int32),
            pltpu.VMEM((SB, D), jnp.float32),
            pltpu.VMEM((SB, D), jnp.float32),
            pltpu.VMEM((SB, D), jnp.float32),
            pltpu.VMEM_SHARED((N, D), jnp.float32),
            pltpu.SemaphoreType.DMA,
            pltpu.SemaphoreType.DMA,
            pltpu.SemaphoreType.DMA,
        ],
    )(_sc_body)
    return f(w, x, ei)


_BN = 2000


def _update_body(p_ref, x_ref, w1_ref, b1_ref, w2_ref, b2_ref, o_ref):
    m = p_ref[0] + p_ref[1]
    u = _ssp(jnp.dot(m, w1_ref[...], preferred_element_type=jnp.float32)
             + b1_ref[...])
    o_ref[...] = (
        x_ref[...]
        + jnp.dot(u, w2_ref[...], preferred_element_type=jnp.float32)
        + b2_ref[...]
    )


def _update_mlp(partials, x, uW1, ub1, uW2, ub2):
    grid = N // _BN
    return pl.pallas_call(
        _update_body,
        grid=(grid,),
        in_specs=[
            pl.BlockSpec((NC, _BN, D), lambda i: (0, i, 0)),
            pl.BlockSpec((_BN, D), lambda i: (i, 0)),
            pl.BlockSpec((D, D), lambda i: (0, 0)),
            pl.BlockSpec((1, D), lambda i: (0, 0)),
            pl.BlockSpec((D, D), lambda i: (0, 0)),
            pl.BlockSpec((1, D), lambda i: (0, 0)),
        ],
        out_specs=pl.BlockSpec((_BN, D), lambda i: (i, 0)),
        out_shape=jax.ShapeDtypeStruct((N, D), jnp.float32),
    )(partials, x, uW1, ub1.reshape(1, D), uW2, ub2.reshape(1, D))


def kernel(x, edge_index, e_ij, fW1, fb1, fW2, fb2, uW1, ub1, uW2, ub2):
    ei = edge_index.astype(jnp.int32)
    w = _filter_mlp(e_ij, fW1, fb1, fW2, fb2)
    partials = _sc_gather_scatter(w, x, ei)
    return _update_mlp(partials, x, uW1, ub1, uW2, ub2)


# speedup vs baseline: 3.8935x; 1.1586x over previous
import functools

import numpy as np

import jax
import jax.numpy as jnp
from jax import lax
from jax.experimental import pallas as pl
from jax.experimental.pallas import tpu as pltpu
from jax.experimental.pallas import tpu_sc as plsc

N = 10000
E = 320000
D = 128
NRBF = 16

NC = 2
NS = 16
NW = NC * NS
SB = 128
TS = E // SB
NSTR = TS // NW
EXTRA = TS - NSTR * NW
ZR = 624
ZR_TAIL = N - ZR * NS

_LOG2 = 0.6931471805599453


def _ssp(t):
    return jnp.maximum(t, 0.0) + jnp.log1p(jnp.exp(-jnp.abs(t))) - _LOG2


_BE = 8000


def _filter_body(e_ref, w1_ref, b1_ref, w2_ref, b2_ref, o_ref):
    t = jnp.dot(e_ref[...], w1_ref[...], preferred_element_type=jnp.float32)
    h = _ssp(t + b1_ref[...])
    o_ref[...] = (
        jnp.dot(h, w2_ref[...], preferred_element_type=jnp.float32)
        + b2_ref[...]
    )


def _filter_mlp(e_ij, fW1, fb1, fW2, fb2):
    grid = E // _BE
    return pl.pallas_call(
        _filter_body,
        grid=(grid,),
        in_specs=[
            pl.BlockSpec((_BE, NRBF), lambda i: (i, 0)),
            pl.BlockSpec((NRBF, D), lambda i: (0, 0)),
            pl.BlockSpec((1, D), lambda i: (0, 0)),
            pl.BlockSpec((D, D), lambda i: (0, 0)),
            pl.BlockSpec((1, D), lambda i: (0, 0)),
        ],
        out_specs=pl.BlockSpec((_BE, D), lambda i: (i, 0)),
        out_shape=jax.ShapeDtypeStruct((E, D), jnp.float32),
    )(e_ij, fW1, fb1.reshape(1, D), fW2, fb2.reshape(1, D))


def _sc_body(w_hbm, x_hbm, ei_hbm, out_hbm,
             src0, src1, dst0, dst1, xg0, xg1, wv, acc_sh,
             sg0, sg1, sw0, ss0, ss1):
    c = lax.axis_index("c")
    s = lax.axis_index("s")
    wid = s * NC + c
    xg = (xg0, xg1)
    srcb = (src0, src1)
    dstb = (dst0, dst1)
    sg = (sg0, sg1)
    ss = (ss0, ss1)
    sbase = wid * NSTR

    def zrow(r, _):
        for j in range(D // 16):
            xg0[r, pl.ds(16 * j, 16)] = jnp.zeros((16,), jnp.float32)
        return 0

    lax.fori_loop(0, SB, zrow, 0)
    r0 = s * ZR
    for k in range(ZR // SB):
        pltpu.sync_copy(xg0, acc_sh.at[pl.ds(r0 + SB * k, SB)])
    rem = ZR % SB
    if rem:
        pltpu.sync_copy(xg0.at[pl.ds(0, rem)],
                        acc_sh.at[pl.ds(r0 + (ZR // SB) * SB, rem)])

    @pl.when(s == 0)
    def _zero_tail():
        pltpu.sync_copy(xg0.at[pl.ds(0, ZR_TAIL)],
                        acc_sh.at[pl.ds(ZR * NS, ZR_TAIL)])

    plsc.subcore_barrier()

    def eb(si):
        return (sbase + si) * SB

    def issue_gather(si, b):
        pltpu.sync_copy(ei_hbm.at[1, pl.ds(eb(si), SB)], dstb[b].at[0])
        pltpu.async_copy(x_hbm.at[dstb[b].at[0]], xg[b], sg[b])
        pltpu.sync_copy(ei_hbm.at[0, pl.ds(eb(si), SB)], srcb[b].at[0])

    def issue_w(si):
        pltpu.async_copy(w_hbm.at[pl.ds(eb(si), SB)], wv, sw0)

    def wait_gather(b):
        pltpu.make_async_copy(x_hbm.at[dstb[b].at[0]], xg[b], sg[b]).wait()

    def wait_w(si):
        pltpu.make_async_copy(w_hbm.at[pl.ds(eb(si), SB)], wv, sw0).wait()

    def compute(b):
        def mrow(rr, _):
            ra = 2 * rr
            rb = 2 * rr + 1
            for j in range(D // 16):
                sl = pl.ds(16 * j, 16)
                xg[b][ra, sl] = wv[ra, sl] * xg[b][ra, sl]
                xg[b][rb, sl] = wv[rb, sl] * xg[b][rb, sl]
            return 0

        lax.fori_loop(0, SB // 2, mrow, 0)

    def scatter_start(b):
        pltpu.async_copy(xg[b], acc_sh.at[srcb[b].at[0]], ss[b], add=True)

    def scatter_wait(b):
        pltpu.make_async_copy(xg[b], acc_sh.at[srcb[b].at[0]], ss[b]).wait()

    issue_gather(0, 0)
    issue_w(0)

    def pair(g, _):
        for b in (0, 1):
            si = 2 * g + b
            wait_gather(b)
            wait_w(si)
            compute(b)
            scatter_start(b)

            @pl.when(si + 1 < NSTR)
            def _next_w():
                issue_w(si + 1)

            @pl.when(si >= 1)
            def _drain():
                scatter_wait(1 - b)

            @pl.when(si + 1 < NSTR)
            def _next_g():
                issue_gather(si + 1, 1 - b)

        return 0

    lax.fori_loop(0, NSTR // 2, pair, 0)
    scatter_wait(1)

    @pl.when(wid < EXTRA)
    def _extra_stream():
        ebx = (NSTR * NW + wid) * SB
        pltpu.sync_copy(ei_hbm.at[1, pl.ds(ebx, SB)], dst0.at[0])
        pltpu.async_copy(x_hbm.at[dst0.at[0]], xg0, sg0).wait()
        pltpu.sync_copy(ei_hbm.at[0, pl.ds(ebx, SB)], src0.at[0])
        pltpu.async_copy(w_hbm.at[pl.ds(ebx, SB)], wv, sw0).wait()
        compute(0)
        scatter_start(0)
        scatter_wait(0)

    plsc.subcore_barrier()

    pltpu.sync_copy(acc_sh.at[pl.ds(r0, ZR)], out_hbm.at[c, pl.ds(r0, ZR)])

    @pl.when(s == 0)
    def _write_tail():
        pltpu.sync_copy(acc_sh.at[pl.ds(ZR * NS, ZR_TAIL)],
                        out_hbm.at[c, pl.ds(ZR * NS, ZR_TAIL)])


def _sc_gather_scatter(w, x, ei):
    f = functools.partial(
        pl.kernel,
        out_type=jax.ShapeDtypeStruct((NC, N, D), jnp.float32),
        mesh=plsc.VectorSubcoreMesh(core_axis_name="c", subcore_axis_name="s"),
        scratch_types=[
            pltpu.VMEM((1, SB), jnp.int32),
            pltpu.VMEM((1, SB), jnp.int32),
            pltpu.VMEM((1, SB), jnp.int32),
            pltpu.VMEM((1, SB), jnp.int32),
            pltpu.VMEM((SB, D), jnp.float32),
            pltpu.VMEM((SB, D), jnp.float32),
            pltpu.VMEM((SB, D), jnp.float32),
            pltpu.VMEM_SHARED((N, D), jnp.float32),
            pltpu.SemaphoreType.DMA,
            pltpu.SemaphoreType.DMA,
            pltpu.SemaphoreType.DMA,
            pltpu.SemaphoreType.DMA,
            pltpu.SemaphoreType.DMA,
        ],
    )(_sc_body)
    return f(w, x, ei)


_BN = 2000


def _update_body(p_ref, x_ref, w1_ref, b1_ref, w2_ref, b2_ref, o_ref):
    m = p_ref[0] + p_ref[1]
    u = _ssp(jnp.dot(m, w1_ref[...], preferred_element_type=jnp.float32)
             + b1_ref[...])
    o_ref[...] = (
        x_ref[...]
        + jnp.dot(u, w2_ref[...], preferred_element_type=jnp.float32)
        + b2_ref[...]
    )


def _update_mlp(partials, x, uW1, ub1, uW2, ub2):
    grid = N // _BN
    return pl.pallas_call(
        _update_body,
        grid=(grid,),
        in_specs=[
            pl.BlockSpec((NC, _BN, D), lambda i: (0, i, 0)),
            pl.BlockSpec((_BN, D), lambda i: (i, 0)),
            pl.BlockSpec((D, D), lambda i: (0, 0)),
            pl.BlockSpec((1, D), lambda i: (0, 0)),
            pl.BlockSpec((D, D), lambda i: (0, 0)),
            pl.BlockSpec((1, D), lambda i: (0, 0)),
        ],
        out_specs=pl.BlockSpec((_BN, D), lambda i: (i, 0)),
        out_shape=jax.ShapeDtypeStruct((N, D), jnp.float32),
    )(partials, x, uW1, ub1.reshape(1, D), uW2, ub2.reshape(1, D))


def kernel(x, edge_index, e_ij, fW1, fb1, fW2, fb2, uW1, ub1, uW2, ub2):
    ei = edge_index.astype(jnp.int32)
    w = _filter_mlp(e_ij, fW1, fb1, fW2, fb2)
    partials = _sc_gather_scatter(w, x, ei)
    return _update_mlp(partials, x, uW1, ub1, uW2, ub2)


# speedup vs baseline: 3.9506x; 1.0147x over previous
import functools

import numpy as np

import jax
import jax.numpy as jnp
from jax import lax
from jax.experimental import pallas as pl
from jax.experimental.pallas import tpu as pltpu
from jax.experimental.pallas import tpu_sc as plsc

N = 10000
E = 320000
D = 128
NRBF = 16

NC = 2
NS = 16
NW = NC * NS
SB = 128
TS = E // SB
NSTR = TS // NW
EXTRA = TS - NSTR * NW
ZR = 624
ZR_TAIL = N - ZR * NS

_LOG2 = 0.6931471805599453


def _ssp(t):
    return jnp.maximum(t, 0.0) + jnp.log1p(jnp.exp(-jnp.abs(t))) - _LOG2


_BE = 8000


def _filter_body(e_ref, w1_ref, b1_ref, w2_ref, b2_ref, o_ref):
    t = jnp.dot(e_ref[...], w1_ref[...], preferred_element_type=jnp.float32)
    h = _ssp(t + b1_ref[...])
    res = (
        jnp.dot(h, w2_ref[...], preferred_element_type=jnp.float32)
        + b2_ref[...]
    ).astype(jnp.bfloat16)
    lo = jax.lax.bitcast_convert_type(res[:, : D // 2], jnp.uint16)
    hi = jax.lax.bitcast_convert_type(res[:, D // 2:], jnp.uint16)
    o_ref[...] = jax.lax.bitcast_convert_type(
        lo.astype(jnp.uint32) | (hi.astype(jnp.uint32) << 16), jnp.int32)


def _filter_mlp(e_ij, fW1, fb1, fW2, fb2):
    grid = E // _BE
    return pl.pallas_call(
        _filter_body,
        grid=(grid,),
        in_specs=[
            pl.BlockSpec((_BE, NRBF), lambda i: (i, 0)),
            pl.BlockSpec((NRBF, D), lambda i: (0, 0)),
            pl.BlockSpec((1, D), lambda i: (0, 0)),
            pl.BlockSpec((D, D), lambda i: (0, 0)),
            pl.BlockSpec((1, D), lambda i: (0, 0)),
        ],
        out_specs=pl.BlockSpec((_BE, D // 2), lambda i: (i, 0)),
        out_shape=jax.ShapeDtypeStruct((E, D // 2), jnp.int32),
    )(e_ij, fW1, fb1.reshape(1, D), fW2, fb2.reshape(1, D))


def _sc_body(w_hbm, x_hbm, ei_hbm, out_hbm,
             src0, src1, dst0, dst1, xg0, xg1, wv, acc_sh,
             sg0, sg1, sw0, ss0, ss1):
    c = lax.axis_index("c")
    s = lax.axis_index("s")
    wid = s * NC + c
    xg = (xg0, xg1)
    srcb = (src0, src1)
    dstb = (dst0, dst1)
    sg = (sg0, sg1)
    ss = (ss0, ss1)
    sbase = wid * NSTR

    def zrow(r, _):
        for j in range(D // 16):
            xg0[r, pl.ds(16 * j, 16)] = jnp.zeros((16,), jnp.float32)
        return 0

    lax.fori_loop(0, SB, zrow, 0)
    r0 = s * ZR
    for k in range(ZR // SB):
        pltpu.sync_copy(xg0, acc_sh.at[pl.ds(r0 + SB * k, SB)])
    rem = ZR % SB
    if rem:
        pltpu.sync_copy(xg0.at[pl.ds(0, rem)],
                        acc_sh.at[pl.ds(r0 + (ZR // SB) * SB, rem)])

    @pl.when(s == 0)
    def _zero_tail():
        pltpu.sync_copy(xg0.at[pl.ds(0, ZR_TAIL)],
                        acc_sh.at[pl.ds(ZR * NS, ZR_TAIL)])

    plsc.subcore_barrier()

    def eb(si):
        return (sbase + si) * SB

    def issue_gather(si, b):
        pltpu.sync_copy(ei_hbm.at[1, pl.ds(eb(si), SB)], dstb[b].at[0])
        pltpu.async_copy(x_hbm.at[dstb[b].at[0]], xg[b], sg[b])
        pltpu.sync_copy(ei_hbm.at[0, pl.ds(eb(si), SB)], srcb[b].at[0])

    def issue_w(si):
        pltpu.async_copy(w_hbm.at[pl.ds(eb(si), SB)], wv, sw0)

    def wait_gather(b):
        pltpu.make_async_copy(x_hbm.at[dstb[b].at[0]], xg[b], sg[b]).wait()

    def wait_w(si):
        pltpu.make_async_copy(w_hbm.at[pl.ds(eb(si), SB)], wv, sw0).wait()

    def compute(b):
        mask = jnp.int32(-65536)

        def mrow(r, _):
            for m in range(D // 32):
                wpk = wv[r, pl.ds(16 * m, 16)]
                sll = pl.ds(16 * m, 16)
                slh = pl.ds(D // 2 + 16 * m, 16)
                wlo = jax.lax.bitcast_convert_type(wpk << 16, jnp.float32)
                whi = jax.lax.bitcast_convert_type(wpk & mask, jnp.float32)
                xg[b][r, sll] = wlo * xg[b][r, sll]
                xg[b][r, slh] = whi * xg[b][r, slh]
            return 0

        lax.fori_loop(0, SB, mrow, 0)

    def scatter_start(b):
        pltpu.async_copy(xg[b], acc_sh.at[srcb[b].at[0]], ss[b], add=True)

    def scatter_wait(b):
        pltpu.make_async_copy(xg[b], acc_sh.at[srcb[b].at[0]], ss[b]).wait()

    issue_gather(0, 0)
    issue_w(0)

    def pair(g, _):
        for b in (0, 1):
            si = 2 * g + b
            wait_gather(b)
            wait_w(si)
            compute(b)
            scatter_start(b)

            @pl.when(si + 1 < NSTR)
            def _next_w():
                issue_w(si + 1)

            @pl.when(si >= 1)
            def _drain():
                scatter_wait(1 - b)

            @pl.when(si + 1 < NSTR)
            def _next_g():
                issue_gather(si + 1, 1 - b)

        return 0

    lax.fori_loop(0, NSTR // 2, pair, 0)
    scatter_wait(1)

    @pl.when(wid < EXTRA)
    def _extra_stream():
        ebx = (NSTR * NW + wid) * SB
        pltpu.sync_copy(ei_hbm.at[1, pl.ds(ebx, SB)], dst0.at[0])
        pltpu.async_copy(x_hbm.at[dst0.at[0]], xg0, sg0).wait()
        pltpu.sync_copy(ei_hbm.at[0, pl.ds(ebx, SB)], src0.at[0])
        pltpu.async_copy(w_hbm.at[pl.ds(ebx, SB)], wv, sw0).wait()
        compute(0)
        scatter_start(0)
        scatter_wait(0)

    plsc.subcore_barrier()

    pltpu.sync_copy(acc_sh.at[pl.ds(r0, ZR)], out_hbm.at[c, pl.ds(r0, ZR)])

    @pl.when(s == 0)
    def _write_tail():
        pltpu.sync_copy(acc_sh.at[pl.ds(ZR * NS, ZR_TAIL)],
                        out_hbm.at[c, pl.ds(ZR * NS, ZR_TAIL)])


def _sc_gather_scatter(w, x, ei):
    f = functools.partial(
        pl.kernel,
        out_type=jax.ShapeDtypeStruct((NC, N, D), jnp.float32),
        mesh=plsc.VectorSubcoreMesh(core_axis_name="c", subcore_axis_name="s"),
        scratch_types=[
            pltpu.VMEM((1, SB), jnp.int32),
            pltpu.VMEM((1, SB), jnp.int32),
            pltpu.VMEM((1, SB), jnp.int32),
            pltpu.VMEM((1, SB), jnp.int32),
            pltpu.VMEM((SB, D), jnp.float32),
            pltpu.VMEM((SB, D), jnp.float32),
            pltpu.VMEM((SB, D // 2), jnp.int32),
            pltpu.VMEM_SHARED((N, D), jnp.float32),
            pltpu.SemaphoreType.DMA,
            pltpu.SemaphoreType.DMA,
            pltpu.SemaphoreType.DMA,
            pltpu.SemaphoreType.DMA,
            pltpu.SemaphoreType.DMA,
        ],
    )(_sc_body)
    return f(w, x, ei)


_BN = 2000


def _update_body(p_ref, x_ref, w1_ref, b1_ref, w2_ref, b2_ref, o_ref):
    m = p_ref[0] + p_ref[1]
    u = _ssp(jnp.dot(m, w1_ref[...], preferred_element_type=jnp.float32)
             + b1_ref[...])
    o_ref[...] = (
        x_ref[...]
        + jnp.dot(u, w2_ref[...], preferred_element_type=jnp.float32)
        + b2_ref[...]
    )


def _update_mlp(partials, x, uW1, ub1, uW2, ub2):
    grid = N // _BN
    return pl.pallas_call(
        _update_body,
        grid=(grid,),
        in_specs=[
            pl.BlockSpec((NC, _BN, D), lambda i: (0, i, 0)),
            pl.BlockSpec((_BN, D), lambda i: (i, 0)),
            pl.BlockSpec((D, D), lambda i: (0, 0)),
            pl.BlockSpec((1, D), lambda i: (0, 0)),
            pl.BlockSpec((D, D), lambda i: (0, 0)),
            pl.BlockSpec((1, D), lambda i: (0, 0)),
        ],
        out_specs=pl.BlockSpec((_BN, D), lambda i: (i, 0)),
        out_shape=jax.ShapeDtypeStruct((N, D), jnp.float32),
    )(partials, x, uW1, ub1.reshape(1, D), uW2, ub2.reshape(1, D))


def kernel(x, edge_index, e_ij, fW1, fb1, fW2, fb2, uW1, ub1, uW2, ub2):
    ei = edge_index.astype(jnp.int32)
    w = _filter_mlp(e_ij, fW1, fb1, fW2, fb2)
    partials = _sc_gather_scatter(w, x, ei)
    return _update_mlp(partials, x, uW1, ub1, uW2, ub2)


# speedup vs baseline: 3.9644x; 1.0035x over previous
import functools

import numpy as np

import jax
import jax.numpy as jnp
from jax import lax
from jax.experimental import pallas as pl
from jax.experimental.pallas import tpu as pltpu
from jax.experimental.pallas import tpu_sc as plsc

N = 10000
E = 320000
D = 128
NRBF = 16

NC = 2
NS = 16
NW = NC * NS
SB = 128
TS = E // SB
NSTR = TS // NW
EXTRA = TS - NSTR * NW
ZR = 624
ZR_TAIL = N - ZR * NS

_LOG2 = 0.6931471805599453


def _ssp(t):
    return jnp.maximum(t, 0.0) + jnp.log1p(jnp.exp(-jnp.abs(t))) - _LOG2


_BE = 8000


def _filter_body(e_ref, w1_ref, b1_ref, w2_ref, b2_ref, o_ref):
    t = jnp.dot(e_ref[...], w1_ref[...], preferred_element_type=jnp.float32)
    h = _ssp(t + b1_ref[...])
    res = (
        jnp.dot(h, w2_ref[...], preferred_element_type=jnp.float32)
        + b2_ref[...]
    ).astype(jnp.bfloat16)
    lo = jax.lax.bitcast_convert_type(res[:, : D // 2], jnp.uint16)
    hi = jax.lax.bitcast_convert_type(res[:, D // 2:], jnp.uint16)
    o_ref[...] = jax.lax.bitcast_convert_type(
        lo.astype(jnp.uint32) | (hi.astype(jnp.uint32) << 16), jnp.int32)


def _filter_mlp(e_ij, fW1, fb1, fW2, fb2):
    grid = E // _BE
    return pl.pallas_call(
        _filter_body,
        grid=(grid,),
        in_specs=[
            pl.BlockSpec((_BE, NRBF), lambda i: (i, 0)),
            pl.BlockSpec((NRBF, D), lambda i: (0, 0)),
            pl.BlockSpec((1, D), lambda i: (0, 0)),
            pl.BlockSpec((D, D), lambda i: (0, 0)),
            pl.BlockSpec((1, D), lambda i: (0, 0)),
        ],
        out_specs=pl.BlockSpec((_BE, D // 2), lambda i: (i, 0)),
        out_shape=jax.ShapeDtypeStruct((E, D // 2), jnp.int32),
    )(e_ij, fW1, fb1.reshape(1, D), fW2, fb2.reshape(1, D))


def _sc_body(w_hbm, x_hbm, ei_hbm, out_hbm,
             src0, src1, dst0, dst1, xg0, xg1, wv, acc_sh,
             sg0, sg1, sw0, ss0, ss1, sdt0, sdt1, ssr0, ssr1):
    c = lax.axis_index("c")
    s = lax.axis_index("s")
    wid = s * NC + c
    xg = (xg0, xg1)
    srcb = (src0, src1)
    dstb = (dst0, dst1)
    sg = (sg0, sg1)
    ss = (ss0, ss1)
    sdt = (sdt0, sdt1)
    ssr = (ssr0, ssr1)
    sbase = wid * NSTR

    def zrow(r, _):
        for j in range(D // 16):
            xg0[r, pl.ds(16 * j, 16)] = jnp.zeros((16,), jnp.float32)
        return 0

    lax.fori_loop(0, SB, zrow, 0)
    r0 = s * ZR
    for k in range(ZR // SB):
        pltpu.sync_copy(xg0, acc_sh.at[pl.ds(r0 + SB * k, SB)])
    rem = ZR % SB
    if rem:
        pltpu.sync_copy(xg0.at[pl.ds(0, rem)],
                        acc_sh.at[pl.ds(r0 + (ZR // SB) * SB, rem)])

    @pl.when(s == 0)
    def _zero_tail():
        pltpu.sync_copy(xg0.at[pl.ds(0, ZR_TAIL)],
                        acc_sh.at[pl.ds(ZR * NS, ZR_TAIL)])

    plsc.subcore_barrier()

    def eb(si):
        return (sbase + si) * SB

    def issue_dst(si, b):
        pltpu.async_copy(ei_hbm.at[1, pl.ds(eb(si), SB)], dstb[b].at[0],
                         sdt[b])

    def wait_dst(si, b):
        pltpu.make_async_copy(ei_hbm.at[1, pl.ds(eb(si), SB)], dstb[b].at[0],
                              sdt[b]).wait()

    def issue_src(si, b):
        pltpu.async_copy(ei_hbm.at[0, pl.ds(eb(si), SB)], srcb[b].at[0],
                         ssr[b])

    def wait_src(si, b):
        pltpu.make_async_copy(ei_hbm.at[0, pl.ds(eb(si), SB)], srcb[b].at[0],
                              ssr[b]).wait()

    def issue_gather(si, b):
        pltpu.async_copy(x_hbm.at[dstb[b].at[0]], xg[b], sg[b])

    def issue_w(si):
        pltpu.async_copy(w_hbm.at[pl.ds(eb(si), SB)], wv, sw0)

    def wait_gather(b):
        pltpu.make_async_copy(x_hbm.at[dstb[b].at[0]], xg[b], sg[b]).wait()

    def wait_w(si):
        pltpu.make_async_copy(w_hbm.at[pl.ds(eb(si), SB)], wv, sw0).wait()

    def compute(b):
        mask = jnp.int32(-65536)

        def mrow(r, _):
            for m in range(D // 32):
                wpk = wv[r, pl.ds(16 * m, 16)]
                sll = pl.ds(16 * m, 16)
                slh = pl.ds(D // 2 + 16 * m, 16)
                wlo = jax.lax.bitcast_convert_type(wpk << 16, jnp.float32)
                whi = jax.lax.bitcast_convert_type(wpk & mask, jnp.float32)
                xg[b][r, sll] = wlo * xg[b][r, sll]
                xg[b][r, slh] = whi * xg[b][r, slh]
            return 0

        lax.fori_loop(0, SB, mrow, 0)

    def scatter_start(b):
        pltpu.async_copy(xg[b], acc_sh.at[srcb[b].at[0]], ss[b], add=True)

    def scatter_wait(b):
        pltpu.make_async_copy(xg[b], acc_sh.at[srcb[b].at[0]], ss[b]).wait()

    issue_dst(0, 0)
    issue_src(0, 0)
    wait_dst(0, 0)
    issue_gather(0, 0)
    issue_w(0)

    def pair(g, _):
        for b in (0, 1):
            si = 2 * g + b
            wait_gather(b)

            @pl.when(si + 1 < NSTR)
            def _next_dst():
                issue_dst(si + 1, 1 - b)

            wait_w(si)
            compute(b)
            wait_src(si, b)
            scatter_start(b)

            @pl.when(si + 1 < NSTR)
            def _next_w():
                issue_w(si + 1)

            @pl.when(si >= 1)
            def _drain():
                scatter_wait(1 - b)

            @pl.when(si + 1 < NSTR)
            def _next_src():
                issue_src(si + 1, 1 - b)

            @pl.when(si + 1 < NSTR)
            def _next_g():
                wait_dst(si + 1, 1 - b)
                issue_gather(si + 1, 1 - b)

        return 0

    lax.fori_loop(0, NSTR // 2, pair, 0)
    scatter_wait(1)

    @pl.when(wid < EXTRA)
    def _extra_stream():
        ebx = (NSTR * NW + wid) * SB
        pltpu.sync_copy(ei_hbm.at[1, pl.ds(ebx, SB)], dst0.at[0])
        pltpu.async_copy(x_hbm.at[dst0.at[0]], xg0, sg0).wait()
        pltpu.sync_copy(ei_hbm.at[0, pl.ds(ebx, SB)], src0.at[0])
        pltpu.async_copy(w_hbm.at[pl.ds(ebx, SB)], wv, sw0).wait()
        compute(0)
        scatter_start(0)
        scatter_wait(0)

    plsc.subcore_barrier()

    pltpu.sync_copy(acc_sh.at[pl.ds(r0, ZR)], out_hbm.at[c, pl.ds(r0, ZR)])

    @pl.when(s == 0)
    def _write_tail():
        pltpu.sync_copy(acc_sh.at[pl.ds(ZR * NS, ZR_TAIL)],
                        out_hbm.at[c, pl.ds(ZR * NS, ZR_TAIL)])


def _sc_gather_scatter(w, x, ei):
    f = functools.partial(
        pl.kernel,
        out_type=jax.ShapeDtypeStruct((NC, N, D), jnp.float32),
        mesh=plsc.VectorSubcoreMesh(core_axis_name="c", subcore_axis_name="s"),
        scratch_types=[
            pltpu.VMEM((1, SB), jnp.int32),
            pltpu.VMEM((1, SB), jnp.int32),
            pltpu.VMEM((1, SB), jnp.int32),
            pltpu.VMEM((1, SB), jnp.int32),
            pltpu.VMEM((SB, D), jnp.float32),
            pltpu.VMEM((SB, D), jnp.float32),
            pltpu.VMEM((SB, D // 2), jnp.int32),
            pltpu.VMEM_SHARED((N, D), jnp.float32),
            pltpu.SemaphoreType.DMA,
            pltpu.SemaphoreType.DMA,
            pltpu.SemaphoreType.DMA,
            pltpu.SemaphoreType.DMA,
            pltpu.SemaphoreType.DMA,
            pltpu.SemaphoreType.DMA,
            pltpu.SemaphoreType.DMA,
            pltpu.SemaphoreType.DMA,
            pltpu.SemaphoreType.DMA,
        ],
    )(_sc_body)
    return f(w, x, ei)


_BN = 2000


def _update_body(p_ref, x_ref, w1_ref, b1_ref, w2_ref, b2_ref, o_ref):
    m = p_ref[0] + p_ref[1]
    u = _ssp(jnp.dot(m, w1_ref[...], preferred_element_type=jnp.float32)
             + b1_ref[...])
    o_ref[...] = (
        x_ref[...]
        + jnp.dot(u, w2_ref[...], preferred_element_type=jnp.float32)
        + b2_ref[...]
    )


def _update_mlp(partials, x, uW1, ub1, uW2, ub2):
    grid = N // _BN
    return pl.pallas_call(
        _update_body,
        grid=(grid,),
        in_specs=[
            pl.BlockSpec((NC, _BN, D), lambda i: (0, i, 0)),
            pl.BlockSpec((_BN, D), lambda i: (i, 0)),
            pl.BlockSpec((D, D), lambda i: (0, 0)),
            pl.BlockSpec((1, D), lambda i: (0, 0)),
            pl.BlockSpec((D, D), lambda i: (0, 0)),
            pl.BlockSpec((1, D), lambda i: (0, 0)),
        ],
        out_specs=pl.BlockSpec((_BN, D), lambda i: (i, 0)),
        out_shape=jax.ShapeDtypeStruct((N, D), jnp.float32),
    )(partials, x, uW1, ub1.reshape(1, D), uW2, ub2.reshape(1, D))


def kernel(x, edge_index, e_ij, fW1, fb1, fW2, fb2, uW1, ub1, uW2, ub2):
    ei = edge_index.astype(jnp.int32)
    w = _filter_mlp(e_ij, fW1, fb1, fW2, fb2)
    partials = _sc_gather_scatter(w, x, ei)
    return _update_mlp(partials, x, uW1, ub1, uW2, ub2)


# speedup vs baseline: 4.0400x; 1.0191x over previous
import functools

import numpy as np

import jax
import jax.numpy as jnp
from jax import lax
from jax.experimental import pallas as pl
from jax.experimental.pallas import tpu as pltpu
from jax.experimental.pallas import tpu_sc as plsc

N = 10000
E = 320000
D = 128
NRBF = 16

NC = 2
NS = 16
NW = NC * NS
SB = 128
TS = E // SB
NSTR = TS // NW
EXTRA = TS - NSTR * NW
ZR = 624
ZR_TAIL = N - ZR * NS

_LOG2 = 0.6931471805599453


def _ssp(t):
    return jnp.maximum(t, 0.0) + jnp.log1p(jnp.exp(-jnp.abs(t))) - _LOG2


_BE = 16000


def _filter_body(e_ref, w1_ref, b1_ref, w2_ref, b2_ref, o_ref):
    t = jnp.dot(e_ref[...], w1_ref[...], preferred_element_type=jnp.float32)
    h = _ssp(t + b1_ref[...])
    res = (
        jnp.dot(h, w2_ref[...], preferred_element_type=jnp.float32)
        + b2_ref[...]
    ).astype(jnp.bfloat16)
    lo = jax.lax.bitcast_convert_type(res[:, : D // 2], jnp.uint16)
    hi = jax.lax.bitcast_convert_type(res[:, D // 2:], jnp.uint16)
    o_ref[...] = jax.lax.bitcast_convert_type(
        lo.astype(jnp.uint32) | (hi.astype(jnp.uint32) << 16), jnp.int32)


def _filter_mlp(e_ij, fW1, fb1, fW2, fb2):
    grid = E // _BE
    return pl.pallas_call(
        _filter_body,
        grid=(grid,),
        in_specs=[
            pl.BlockSpec((_BE, NRBF), lambda i: (i, 0)),
            pl.BlockSpec((NRBF, D), lambda i: (0, 0)),
            pl.BlockSpec((1, D), lambda i: (0, 0)),
            pl.BlockSpec((D, D), lambda i: (0, 0)),
            pl.BlockSpec((1, D), lambda i: (0, 0)),
        ],
        out_specs=pl.BlockSpec((_BE, D // 2), lambda i: (i, 0)),
        out_shape=jax.ShapeDtypeStruct((E, D // 2), jnp.int32),
    )(e_ij, fW1, fb1.reshape(1, D), fW2, fb2.reshape(1, D))


def _sc_body(w_hbm, x_hbm, ei_hbm, out_hbm,
             src0, src1, dst0, dst1, xg0, xg1, wv, acc_sh,
             sg0, sg1, sw0, ss0, ss1, sdt0, sdt1, ssr0, ssr1):
    c = lax.axis_index("c")
    s = lax.axis_index("s")
    wid = s * NC + c
    xg = (xg0, xg1)
    srcb = (src0, src1)
    dstb = (dst0, dst1)
    sg = (sg0, sg1)
    ss = (ss0, ss1)
    sdt = (sdt0, sdt1)
    ssr = (ssr0, ssr1)
    sbase = wid * NSTR

    def zrow(r, _):
        for j in range(D // 16):
            xg0[r, pl.ds(16 * j, 16)] = jnp.zeros((16,), jnp.float32)
        return 0

    lax.fori_loop(0, SB, zrow, 0)
    r0 = s * ZR
    for k in range(ZR // SB):
        pltpu.sync_copy(xg0, acc_sh.at[pl.ds(r0 + SB * k, SB)])
    rem = ZR % SB
    if rem:
        pltpu.sync_copy(xg0.at[pl.ds(0, rem)],
                        acc_sh.at[pl.ds(r0 + (ZR // SB) * SB, rem)])

    @pl.when(s == 0)
    def _zero_tail():
        pltpu.sync_copy(xg0.at[pl.ds(0, ZR_TAIL)],
                        acc_sh.at[pl.ds(ZR * NS, ZR_TAIL)])

    plsc.subcore_barrier()

    def eb(si):
        return (sbase + si) * SB

    def issue_dst(si, b):
        pltpu.async_copy(ei_hbm.at[1, pl.ds(eb(si), SB)], dstb[b].at[0],
                         sdt[b])

    def wait_dst(si, b):
        pltpu.make_async_copy(ei_hbm.at[1, pl.ds(eb(si), SB)], dstb[b].at[0],
                              sdt[b]).wait()

    def issue_src(si, b):
        pltpu.async_copy(ei_hbm.at[0, pl.ds(eb(si), SB)], srcb[b].at[0],
                         ssr[b])

    def wait_src(si, b):
        pltpu.make_async_copy(ei_hbm.at[0, pl.ds(eb(si), SB)], srcb[b].at[0],
                              ssr[b]).wait()

    def issue_gather(si, b):
        pltpu.async_copy(x_hbm.at[dstb[b].at[0]], xg[b], sg[b])

    def issue_w(si):
        pltpu.async_copy(w_hbm.at[pl.ds(eb(si), SB)], wv, sw0)

    def wait_gather(b):
        pltpu.make_async_copy(x_hbm.at[dstb[b].at[0]], xg[b], sg[b]).wait()

    def wait_w(si):
        pltpu.make_async_copy(w_hbm.at[pl.ds(eb(si), SB)], wv, sw0).wait()

    def compute(b):
        mask = jnp.int32(-65536)

        def mrow(r, _):
            for m in range(D // 32):
                wpk = wv[r, pl.ds(16 * m, 16)]
                sll = pl.ds(16 * m, 16)
                slh = pl.ds(D // 2 + 16 * m, 16)
                wlo = jax.lax.bitcast_convert_type(wpk << 16, jnp.float32)
                whi = jax.lax.bitcast_convert_type(wpk & mask, jnp.float32)
                xg[b][r, sll] = wlo * xg[b][r, sll]
                xg[b][r, slh] = whi * xg[b][r, slh]
            return 0

        lax.fori_loop(0, SB, mrow, 0)

    def scatter_start(b):
        pltpu.async_copy(xg[b], acc_sh.at[srcb[b].at[0]], ss[b], add=True)

    def scatter_wait(b):
        pltpu.make_async_copy(xg[b], acc_sh.at[srcb[b].at[0]], ss[b]).wait()

    issue_dst(0, 0)
    issue_src(0, 0)
    wait_dst(0, 0)
    issue_gather(0, 0)
    issue_w(0)

    def pair(g, _):
        for b in (0, 1):
            si = 2 * g + b
            wait_gather(b)

            @pl.when(si + 1 < NSTR)
            def _next_dst():
                issue_dst(si + 1, 1 - b)

            wait_w(si)
            compute(b)
            wait_src(si, b)
            scatter_start(b)

            @pl.when(si + 1 < NSTR)
            def _next_w():
                issue_w(si + 1)

            @pl.when(si >= 1)
            def _drain():
                scatter_wait(1 - b)

            @pl.when(si + 1 < NSTR)
            def _next_src():
                issue_src(si + 1, 1 - b)

            @pl.when(si + 1 < NSTR)
            def _next_g():
                wait_dst(si + 1, 1 - b)
                issue_gather(si + 1, 1 - b)

        return 0

    lax.fori_loop(0, NSTR // 2, pair, 0)
    scatter_wait(1)

    @pl.when(wid < EXTRA)
    def _extra_stream():
        ebx = (NSTR * NW + wid) * SB
        pltpu.sync_copy(ei_hbm.at[1, pl.ds(ebx, SB)], dst0.at[0])
        pltpu.async_copy(x_hbm.at[dst0.at[0]], xg0, sg0).wait()
        pltpu.sync_copy(ei_hbm.at[0, pl.ds(ebx, SB)], src0.at[0])
        pltpu.async_copy(w_hbm.at[pl.ds(ebx, SB)], wv, sw0).wait()
        compute(0)
        scatter_start(0)
        scatter_wait(0)

    plsc.subcore_barrier()

    pltpu.sync_copy(acc_sh.at[pl.ds(r0, ZR)], out_hbm.at[c, pl.ds(r0, ZR)])

    @pl.when(s == 0)
    def _write_tail():
        pltpu.sync_copy(acc_sh.at[pl.ds(ZR * NS, ZR_TAIL)],
                        out_hbm.at[c, pl.ds(ZR * NS, ZR_TAIL)])


def _sc_gather_scatter(w, x, ei):
    f = functools.partial(
        pl.kernel,
        out_type=jax.ShapeDtypeStruct((NC, N, D), jnp.float32),
        mesh=plsc.VectorSubcoreMesh(core_axis_name="c", subcore_axis_name="s"),
        scratch_types=[
            pltpu.VMEM((1, SB), jnp.int32),
            pltpu.VMEM((1, SB), jnp.int32),
            pltpu.VMEM((1, SB), jnp.int32),
            pltpu.VMEM((1, SB), jnp.int32),
            pltpu.VMEM((SB, D), jnp.float32),
            pltpu.VMEM((SB, D), jnp.float32),
            pltpu.VMEM((SB, D // 2), jnp.int32),
            pltpu.VMEM_SHARED((N, D), jnp.float32),
            pltpu.SemaphoreType.DMA,
            pltpu.SemaphoreType.DMA,
            pltpu.SemaphoreType.DMA,
            pltpu.SemaphoreType.DMA,
            pltpu.SemaphoreType.DMA,
            pltpu.SemaphoreType.DMA,
            pltpu.SemaphoreType.DMA,
            pltpu.SemaphoreType.DMA,
            pltpu.SemaphoreType.DMA,
        ],
    )(_sc_body)
    return f(w, x, ei)


_BN = 5000


def _update_body(p_ref, x_ref, w1_ref, b1_ref, w2_ref, b2_ref, o_ref):
    m = p_ref[0] + p_ref[1]
    u = _ssp(jnp.dot(m, w1_ref[...], preferred_element_type=jnp.float32)
             + b1_ref[...])
    o_ref[...] = (
        x_ref[...]
        + jnp.dot(u, w2_ref[...], preferred_element_type=jnp.float32)
        + b2_ref[...]
    )


def _update_mlp(partials, x, uW1, ub1, uW2, ub2):
    grid = N // _BN
    return pl.pallas_call(
        _update_body,
        grid=(grid,),
        in_specs=[
            pl.BlockSpec((NC, _BN, D), lambda i: (0, i, 0)),
            pl.BlockSpec((_BN, D), lambda i: (i, 0)),
            pl.BlockSpec((D, D), lambda i: (0, 0)),
            pl.BlockSpec((1, D), lambda i: (0, 0)),
            pl.BlockSpec((D, D), lambda i: (0, 0)),
            pl.BlockSpec((1, D), lambda i: (0, 0)),
        ],
        out_specs=pl.BlockSpec((_BN, D), lambda i: (i, 0)),
        out_shape=jax.ShapeDtypeStruct((N, D), jnp.float32),
    )(partials, x, uW1, ub1.reshape(1, D), uW2, ub2.reshape(1, D))


def kernel(x, edge_index, e_ij, fW1, fb1, fW2, fb2, uW1, ub1, uW2, ub2):
    ei = edge_index.astype(jnp.int32)
    w = _filter_mlp(e_ij, fW1, fb1, fW2, fb2)
    partials = _sc_gather_scatter(w, x, ei)
    return _update_mlp(partials, x, uW1, ub1, uW2, ub2)
